# Initial kernel scaffold; baseline (speedup 1.0000x reference)
import jax
import jax.numpy as jnp
from jax.experimental import pallas as pl


def kernel(input, adj, weight, bias):
    raise NotImplementedError("write your pallas kernel here")


# speedup vs baseline: 1.0143x; 1.0143x over previous
import jax
import jax.numpy as jnp
from jax.experimental import pallas as pl
from jax.experimental.pallas import tpu as pltpu

_BETA = 0.001
_BM = 200


def _support_body(x_ref, w_ref, out_ref):
    x = x_ref[...].astype(jnp.bfloat16)
    w = w_ref[...].astype(jnp.bfloat16)
    out_ref[...] = jnp.dot(
        x, w, preferred_element_type=jnp.float32
    ).astype(jnp.bfloat16)


def _agg_body(adj_ref, sup_ref, x_ref, bias_ref, out_ref):
    a = adj_ref[...].astype(jnp.bfloat16)
    acc = jnp.dot(a, sup_ref[...], preferred_element_type=jnp.float32)
    out = _BETA * x_ref[...] + (1.0 - _BETA) * acc
    norm = jnp.sqrt(jnp.sum(out * out, axis=1, keepdims=True))
    out = out / jnp.maximum(norm, 1e-12)
    out_ref[...] = out + bias_ref[...]


def kernel(input, adj, weight, bias):
    n, d = input.shape
    sup = pl.pallas_call(
        _support_body,
        out_shape=jax.ShapeDtypeStruct((n, d), jnp.bfloat16),
    )(input, weight)

    bias2d = bias.reshape(1, d)
    bm = _BM
    out = pl.pallas_call(
        _agg_body,
        grid=(n // bm,),
        in_specs=[
            pl.BlockSpec((bm, n), lambda m: (m, 0)),
            pl.BlockSpec((n, d), lambda m: (0, 0)),
            pl.BlockSpec((bm, d), lambda m: (m, 0)),
            pl.BlockSpec((1, d), lambda m: (0, 0)),
        ],
        out_specs=pl.BlockSpec((bm, d), lambda m: (m, 0)),
        out_shape=jax.ShapeDtypeStruct((n, d), jnp.float32),
        compiler_params=pltpu.CompilerParams(
            dimension_semantics=("parallel",),
        ),
    )(adj, sup, input, bias2d)
    return out


# speedup vs baseline: 1.0172x; 1.0029x over previous
import jax
import jax.numpy as jnp
from jax.experimental import pallas as pl
from jax.experimental.pallas import tpu as pltpu

_BETA = 0.001
_BM = 400


def _support_body(x_ref, w_ref, out_ref):
    x = x_ref[...].astype(jnp.bfloat16)
    w = w_ref[...].astype(jnp.bfloat16)
    out_ref[...] = jnp.dot(
        x, w, preferred_element_type=jnp.float32
    ).astype(jnp.bfloat16)


def _agg_body(adj_ref, sup_ref, x_ref, bias_ref, out_ref):
    a = adj_ref[...].astype(jnp.bfloat16)
    acc = jnp.dot(a, sup_ref[...], preferred_element_type=jnp.float32)
    out = _BETA * x_ref[...] + (1.0 - _BETA) * acc
    norm = jnp.sqrt(jnp.sum(out * out, axis=1, keepdims=True))
    out = out / jnp.maximum(norm, 1e-12)
    out_ref[...] = out + bias_ref[...]


def kernel(input, adj, weight, bias):
    n, d = input.shape
    sup = pl.pallas_call(
        _support_body,
        out_shape=jax.ShapeDtypeStruct((n, d), jnp.bfloat16),
    )(input, weight)

    bias2d = bias.reshape(1, d)
    bm = _BM
    out = pl.pallas_call(
        _agg_body,
        grid=(n // bm,),
        in_specs=[
            pl.BlockSpec((bm, n), lambda m: (m, 0)),
            pl.BlockSpec((n, d), lambda m: (0, 0)),
            pl.BlockSpec((bm, d), lambda m: (m, 0)),
            pl.BlockSpec((1, d), lambda m: (0, 0)),
        ],
        out_specs=pl.BlockSpec((bm, d), lambda m: (m, 0)),
        out_shape=jax.ShapeDtypeStruct((n, d), jnp.float32),
        compiler_params=pltpu.CompilerParams(
            dimension_semantics=("parallel",),
        ),
    )(adj, sup, input, bias2d)
    return out


# speedup vs baseline: 1.0886x; 1.0702x over previous
import jax
import jax.numpy as jnp
from jax.experimental import pallas as pl
from jax.experimental.pallas import tpu as pltpu

_BETA = 0.001
_BM = 400


def _body(x_ref, w_ref, bias_ref, adj_ref, out_ref, sup_ref):
    i = pl.program_id(0)

    @pl.when(i == 0)
    def _compute_support():
        xb = x_ref[...].astype(jnp.bfloat16)
        wb = w_ref[...].astype(jnp.bfloat16)
        sup_ref[...] = jnp.dot(
            xb, wb, preferred_element_type=jnp.float32
        ).astype(jnp.bfloat16)

    a = adj_ref[...].astype(jnp.bfloat16)
    acc = jnp.dot(a, sup_ref[...], preferred_element_type=jnp.float32)
    x_blk = x_ref[pl.ds(i * _BM, _BM), :]
    out = _BETA * x_blk + (1.0 - _BETA) * acc
    norm = jnp.sqrt(jnp.sum(out * out, axis=1, keepdims=True))
    out = out / jnp.maximum(norm, 1e-12)
    out_ref[...] = out + bias_ref[...]


def kernel(input, adj, weight, bias):
    n, d = input.shape
    bm = _BM
    out = pl.pallas_call(
        _body,
        grid=(n // bm,),
        in_specs=[
            pl.BlockSpec((n, d), lambda m: (0, 0)),
            pl.BlockSpec((d, d), lambda m: (0, 0)),
            pl.BlockSpec((1, d), lambda m: (0, 0)),
            pl.BlockSpec((bm, n), lambda m: (m, 0)),
        ],
        out_specs=pl.BlockSpec((bm, d), lambda m: (m, 0)),
        out_shape=jax.ShapeDtypeStruct((n, d), jnp.float32),
        scratch_shapes=[pltpu.VMEM((n, d), jnp.bfloat16)],
        compiler_params=pltpu.CompilerParams(
            dimension_semantics=("arbitrary",),
        ),
    )(input, weight, bias.reshape(1, d), adj)
    return out
